# baseline (speedup 1.0000x reference)
import functools

import jax
import jax.numpy as jnp
from jax import lax
from jax.experimental import pallas as pl
from jax.experimental.pallas import tpu as pltpu
from jax.experimental.pallas import tpu_sc as plsc

NC = 2
NS = 16
NW = NC * NS
CH = 128
CL = 16


def _cdiv(a, b):
  return (a + b - 1) // b


def _make_sc_scatter(n_pad, h, k, with_counts):
  rpt = n_pad // NS
  assert k % 2 == 0
  np_pairs = k // 2

  def body(*refs):
    if with_counts:
      (y_hbm, src_hbm, dst_hbm, zrow_hbm, zcnt_hbm, ones_hbm,
       acc_hbm, cnt_hbm,
       src_v, dst_v, rows0_v, rows1_v, ones_v, acc_sh, cnt_sh,
       g0, g1, s0, s1, csem) = refs
    else:
      (y_hbm, src_hbm, dst_hbm, zrow_hbm,
       acc_hbm,
       src_v, dst_v, rows0_v, rows1_v, acc_sh,
       g0, g1, s0, s1) = refs

    c = lax.axis_index("c")
    s = lax.axis_index("s")
    wid = c * NS + s
    r0 = s * rpt

    pltpu.sync_copy(zrow_hbm, acc_sh.at[pl.ds(r0, rpt)])
    if with_counts:
      pltpu.sync_copy(zcnt_hbm, cnt_sh.at[pl.ds(r0, rpt)])
      pltpu.sync_copy(ones_hbm, ones_v)

    pltpu.sync_copy(src_hbm.at[wid], src_v)
    pltpu.sync_copy(dst_hbm.at[wid], dst_v)
    plsc.subcore_barrier()

    def pair(p, carry):
      j0 = 2 * p
      j1 = j0 + 1
      cp0 = pltpu.async_copy(y_hbm.at[src_v.at[j0]], rows0_v, g0)
      cp1 = pltpu.async_copy(y_hbm.at[src_v.at[j1]], rows1_v, g1)
      cp0.wait()
      pltpu.sync_copy(rows0_v, acc_sh.at[dst_v.at[j0]], add=True)
      if with_counts:
        pltpu.sync_copy(ones_v, cnt_sh.at[dst_v.at[j0]], add=True)
      cp1.wait()
      pltpu.sync_copy(rows1_v, acc_sh.at[dst_v.at[j1]], add=True)
      if with_counts:
        pltpu.sync_copy(ones_v, cnt_sh.at[dst_v.at[j1]], add=True)
      return carry

    lax.fori_loop(0, np_pairs, pair, 0)
    plsc.subcore_barrier()

    pltpu.sync_copy(acc_sh.at[pl.ds(r0, rpt)], acc_hbm.at[c, pl.ds(r0, rpt)])
    if with_counts:
      pltpu.sync_copy(cnt_sh.at[pl.ds(r0, rpt)], cnt_hbm.at[c, pl.ds(r0, rpt)])

  out_type = [jax.ShapeDtypeStruct((NC, n_pad, h), jnp.float32)]
  scratch = [
      pltpu.VMEM((k, CH), jnp.int32),
      pltpu.VMEM((k, CH), jnp.int32),
      pltpu.VMEM((CH, h), jnp.float32),
      pltpu.VMEM((CH, h), jnp.float32),
  ]
  if with_counts:
    out_type.append(jax.ShapeDtypeStruct((NC, n_pad, CL), jnp.float32))
    scratch.append(pltpu.VMEM((CH, CL), jnp.float32))
  scratch.append(pltpu.VMEM_SHARED((n_pad, h), jnp.float32))
  if with_counts:
    scratch.append(pltpu.VMEM_SHARED((n_pad, CL), jnp.float32))
  scratch.extend([pltpu.SemaphoreType.DMA] * 4)
  if with_counts:
    scratch.append(pltpu.SemaphoreType.DMA)

  mesh = plsc.VectorSubcoreMesh(
      core_axis_name="c", subcore_axis_name="s",
      num_cores=NC, num_subcores=NS)
  return pl.kernel(
      body, out_type=out_type, mesh=mesh, scratch_types=scratch,
      compiler_params=pltpu.CompilerParams(use_tc_tiling_on_sc=False))


def _tc_in_body(x_ref, wl_ref, wr_ref, y_ref, r_ref):
  xb = x_ref[...]
  y_ref[...] = jnp.dot(xb, wl_ref[...], preferred_element_type=jnp.float32)
  r_ref[...] = jnp.dot(xb, wr_ref[...], preferred_element_type=jnp.float32)


def _tc_mid_body(a0_ref, a1_ref, c0_ref, c1_ref, r_ref, b_ref,
                 wl_ref, wr_ref, y_ref, rr_ref):
  cnt = c0_ref[:, 0:1] + c1_ref[:, 0:1]
  inv = 1.0 / jnp.maximum(cnt, 1.0)
  h = jnp.maximum((a0_ref[...] + a1_ref[...]) * inv + b_ref[...] + r_ref[...],
                  0.0)
  y_ref[...] = jnp.dot(h, wl_ref[...], preferred_element_type=jnp.float32)
  rr_ref[...] = jnp.dot(h, wr_ref[...], preferred_element_type=jnp.float32)


def _tc_out_body(a0_ref, a1_ref, c0_ref, c1_ref, r_ref, b_ref,
                 wh_ref, bh_ref, h_ref, log_ref):
  cnt = c0_ref[:, 0:1] + c1_ref[:, 0:1]
  inv = 1.0 / jnp.maximum(cnt, 1.0)
  h = jnp.maximum((a0_ref[...] + a1_ref[...]) * inv + b_ref[...] + r_ref[...],
                  0.0)
  h_ref[...] = h
  log_ref[...] = (
      jnp.dot(h, wh_ref[...], preferred_element_type=jnp.float32)
      + bh_ref[...])


def kernel(x, edge_index, W1_l, b1_l, W1_r, W2_l, b2_l, W2_r, Wh, bh):
  n, d = x.shape
  h = W1_l.shape[1]
  out = Wh.shape[1]
  e = edge_index.shape[1]

  n_pad = _cdiv(n + 1, NS * 8) * NS * 8
  k = 2 * _cdiv(_cdiv(e, NW), 2 * CH)
  e_pad = NW * k * CH

  src = edge_index[0].astype(jnp.int32)
  dst = edge_index[1].astype(jnp.int32)
  src3 = jnp.pad(src, (0, e_pad - e)).reshape(NW, k, CH)
  dst3 = jnp.pad(dst, (0, e_pad - e), constant_values=n).reshape(NW, k, CH)

  rpt = n_pad // NS
  zrow = jnp.zeros((rpt, h), jnp.float32)
  zcnt = jnp.zeros((rpt, CL), jnp.float32)
  ones = jnp.ones((CH, CL), jnp.float32)

  sc_scatter1 = _make_sc_scatter(n_pad, h, k, with_counts=True)
  sc_scatter2 = _make_sc_scatter(n_pad, h, k, with_counts=False)

  tc_in = pl.pallas_call(
      _tc_in_body,
      out_shape=[jax.ShapeDtypeStruct((n, h), jnp.float32),
                 jax.ShapeDtypeStruct((n, h), jnp.float32)])
  tc_mid = pl.pallas_call(
      _tc_mid_body,
      out_shape=[jax.ShapeDtypeStruct((n, h), jnp.float32),
                 jax.ShapeDtypeStruct((n, h), jnp.float32)])
  tc_out = pl.pallas_call(
      _tc_out_body,
      out_shape=[jax.ShapeDtypeStruct((n, h), jnp.float32),
                 jax.ShapeDtypeStruct((n, out), jnp.float32)])

  y1, r1 = tc_in(x, W1_l, W1_r)
  acc1, cnt = sc_scatter1(y1, src3, dst3, zrow, zcnt, ones)
  c0, c1 = cnt[0, :n], cnt[1, :n]
  y2, r2 = tc_mid(acc1[0, :n], acc1[1, :n], c0, c1, r1,
                  b1_l.reshape(1, h), W2_l, W2_r)
  (acc2,) = sc_scatter2(y2, src3, dst3, zrow)
  h2, logits = tc_out(acc2[0, :n], acc2[1, :n], c0, c1, r2,
                      b2_l.reshape(1, h), Wh, bh.reshape(1, out))
  return (logits, h2)


# speedup vs baseline: 1.7995x; 1.7995x over previous
import functools

import jax
import jax.numpy as jnp
from jax import lax
from jax.experimental import pallas as pl
from jax.experimental.pallas import tpu as pltpu
from jax.experimental.pallas import tpu_sc as plsc

NC = 2
NS = 16
NW = NC * NS
CH = 128
CL = 16


def _cdiv(a, b):
  return (a + b - 1) // b


def _make_sc_scatter(n, n_pad, h, k, with_counts):
  rpt = n_pad // NS
  assert n % NS == 0

  def body(*refs):
    if with_counts:
      (y_hbm, src_hbm, dst_hbm, zrow_hbm, zcnt_hbm, ones_hbm,
       acc_hbm, cnt_hbm,
       src_v, dst_v, rows0_v, rows1_v, ones_v, y_sh, acc_sh, cnt_sh,
       g0, g1, s0, s1, csem) = refs
    else:
      (y_hbm, src_hbm, dst_hbm, zrow_hbm,
       acc_hbm,
       src_v, dst_v, rows0_v, rows1_v, y_sh, acc_sh,
       g0, g1, s0, s1) = refs

    c = lax.axis_index("c")
    s = lax.axis_index("s")
    wid = c * NS + s
    r0 = s * rpt

    n_rows = y_hbm.shape[0]
    ypt = n_rows // NS
    pltpu.sync_copy(y_hbm.at[pl.ds(s * ypt, ypt)], y_sh.at[pl.ds(s * ypt, ypt)])
    pltpu.sync_copy(zrow_hbm, acc_sh.at[pl.ds(r0, rpt)])
    if with_counts:
      pltpu.sync_copy(zcnt_hbm, cnt_sh.at[pl.ds(r0, rpt)])
      pltpu.sync_copy(ones_hbm, ones_v)

    pltpu.sync_copy(src_hbm.at[wid], src_v)
    pltpu.sync_copy(dst_hbm.at[wid], dst_v)
    plsc.subcore_barrier()

    def step(j, carry):
      pltpu.async_copy(y_sh.at[src_v.at[j]], rows0_v, g0).wait()
      pltpu.sync_copy(rows0_v, acc_sh.at[dst_v.at[j]], add=True)
      if with_counts:
        pltpu.sync_copy(ones_v, cnt_sh.at[dst_v.at[j]], add=True)
      return carry

    lax.fori_loop(0, k, step, 0)
    plsc.subcore_barrier()

    pltpu.sync_copy(acc_sh.at[pl.ds(r0, rpt)], acc_hbm.at[c, pl.ds(r0, rpt)])
    if with_counts:
      pltpu.sync_copy(cnt_sh.at[pl.ds(r0, rpt)], cnt_hbm.at[c, pl.ds(r0, rpt)])

  out_type = [jax.ShapeDtypeStruct((NC, n_pad, h), jnp.float32)]
  scratch = [
      pltpu.VMEM((k, CH), jnp.int32),
      pltpu.VMEM((k, CH), jnp.int32),
      pltpu.VMEM((CH, h), jnp.float32),
      pltpu.VMEM((CH, h), jnp.float32),
  ]
  if with_counts:
    out_type.append(jax.ShapeDtypeStruct((NC, n_pad, CL), jnp.float32))
    scratch.append(pltpu.VMEM((CH, CL), jnp.float32))
  scratch.append(pltpu.VMEM_SHARED((n, h), jnp.float32))
  scratch.append(pltpu.VMEM_SHARED((n_pad, h), jnp.float32))
  if with_counts:
    scratch.append(pltpu.VMEM_SHARED((n_pad, CL), jnp.float32))
  scratch.extend([pltpu.SemaphoreType.DMA] * 4)
  if with_counts:
    scratch.append(pltpu.SemaphoreType.DMA)

  mesh = plsc.VectorSubcoreMesh(
      core_axis_name="c", subcore_axis_name="s",
      num_cores=NC, num_subcores=NS)
  return pl.kernel(
      body, out_type=out_type, mesh=mesh, scratch_types=scratch,
      compiler_params=pltpu.CompilerParams(use_tc_tiling_on_sc=False))


def _tc_in_body(x_ref, wl_ref, wr_ref, y_ref, r_ref):
  xb = x_ref[...]
  y_ref[...] = jnp.dot(xb, wl_ref[...], preferred_element_type=jnp.float32)
  r_ref[...] = jnp.dot(xb, wr_ref[...], preferred_element_type=jnp.float32)


def _tc_mid_body(a0_ref, a1_ref, c0_ref, c1_ref, r_ref, b_ref,
                 wl_ref, wr_ref, y_ref, rr_ref):
  cnt = c0_ref[:, 0:1] + c1_ref[:, 0:1]
  inv = 1.0 / jnp.maximum(cnt, 1.0)
  h = jnp.maximum((a0_ref[...] + a1_ref[...]) * inv + b_ref[...] + r_ref[...],
                  0.0)
  y_ref[...] = jnp.dot(h, wl_ref[...], preferred_element_type=jnp.float32)
  rr_ref[...] = jnp.dot(h, wr_ref[...], preferred_element_type=jnp.float32)


def _tc_out_body(a0_ref, a1_ref, c0_ref, c1_ref, r_ref, b_ref,
                 wh_ref, bh_ref, h_ref, log_ref):
  cnt = c0_ref[:, 0:1] + c1_ref[:, 0:1]
  inv = 1.0 / jnp.maximum(cnt, 1.0)
  h = jnp.maximum((a0_ref[...] + a1_ref[...]) * inv + b_ref[...] + r_ref[...],
                  0.0)
  h_ref[...] = h
  log_ref[...] = (
      jnp.dot(h, wh_ref[...], preferred_element_type=jnp.float32)
      + bh_ref[...])


def kernel(x, edge_index, W1_l, b1_l, W1_r, W2_l, b2_l, W2_r, Wh, bh):
  n, d = x.shape
  h = W1_l.shape[1]
  out = Wh.shape[1]
  e = edge_index.shape[1]

  n_pad = _cdiv(n + 1, NS * 8) * NS * 8
  k = 2 * _cdiv(_cdiv(e, NW), 2 * CH)
  e_pad = NW * k * CH

  src = edge_index[0].astype(jnp.int32)
  dst = edge_index[1].astype(jnp.int32)
  src3 = jnp.pad(src, (0, e_pad - e)).reshape(NW, k, CH)
  dst3 = jnp.pad(dst, (0, e_pad - e), constant_values=n).reshape(NW, k, CH)

  rpt = n_pad // NS
  zrow = jnp.zeros((rpt, h), jnp.float32)
  zcnt = jnp.zeros((rpt, CL), jnp.float32)
  ones = jnp.ones((CH, CL), jnp.float32)

  sc_scatter1 = _make_sc_scatter(n, n_pad, h, k, with_counts=True)
  sc_scatter2 = _make_sc_scatter(n, n_pad, h, k, with_counts=False)

  tc_in = pl.pallas_call(
      _tc_in_body,
      out_shape=[jax.ShapeDtypeStruct((n, h), jnp.float32),
                 jax.ShapeDtypeStruct((n, h), jnp.float32)])
  tc_mid = pl.pallas_call(
      _tc_mid_body,
      out_shape=[jax.ShapeDtypeStruct((n, h), jnp.float32),
                 jax.ShapeDtypeStruct((n, h), jnp.float32)])
  tc_out = pl.pallas_call(
      _tc_out_body,
      out_shape=[jax.ShapeDtypeStruct((n, h), jnp.float32),
                 jax.ShapeDtypeStruct((n, out), jnp.float32)])

  y1, r1 = tc_in(x, W1_l, W1_r)
  acc1, cnt = sc_scatter1(y1, src3, dst3, zrow, zcnt, ones)
  c0, c1 = cnt[0, :n], cnt[1, :n]
  y2, r2 = tc_mid(acc1[0, :n], acc1[1, :n], c0, c1, r1,
                  b1_l.reshape(1, h), W2_l, W2_r)
  (acc2,) = sc_scatter2(y2, src3, dst3, zrow)
  h2, logits = tc_out(acc2[0, :n], acc2[1, :n], c0, c1, r2,
                      b2_l.reshape(1, h), Wh, bh.reshape(1, out))
  return (logits, h2)


# speedup vs baseline: 1.9195x; 1.0667x over previous
import functools

import jax
import jax.numpy as jnp
from jax import lax
from jax.experimental import pallas as pl
from jax.experimental.pallas import tpu as pltpu
from jax.experimental.pallas import tpu_sc as plsc

NC = 2
NS = 16
NW = NC * NS
CH = 128
CL = 16


def _cdiv(a, b):
  return (a + b - 1) // b


def _make_sc_scatter(n, n_pad, h, k, with_counts):
  rpt = n_pad // NS
  assert n % NS == 0

  def body(*refs):
    if with_counts:
      (y_hbm, src_hbm, dst_hbm, zrow_hbm, zcnt_hbm, ones_hbm,
       acc_hbm, cnt_hbm,
       src_v, dst_v, rows0_v, rows1_v, ones_v, y_sh, acc_sh, cnt_sh,
       g0, g1, s0, s1, csem) = refs
    else:
      (y_hbm, src_hbm, dst_hbm, zrow_hbm,
       acc_hbm,
       src_v, dst_v, rows0_v, rows1_v, y_sh, acc_sh,
       g0, g1, s0, s1) = refs

    c = lax.axis_index("c")
    s = lax.axis_index("s")
    wid = c * NS + s
    r0 = s * rpt

    n_rows = y_hbm.shape[0]
    ypt = n_rows // NS
    pltpu.sync_copy(y_hbm.at[pl.ds(s * ypt, ypt)], y_sh.at[pl.ds(s * ypt, ypt)])
    pltpu.sync_copy(zrow_hbm, acc_sh.at[pl.ds(r0, rpt)])
    if with_counts:
      pltpu.sync_copy(zcnt_hbm, cnt_sh.at[pl.ds(r0, rpt)])
      pltpu.sync_copy(ones_hbm, ones_v)

    pltpu.sync_copy(src_hbm.at[wid], src_v)
    pltpu.sync_copy(dst_hbm.at[wid], dst_v)
    plsc.subcore_barrier()

    def pair(p, carry):
      j0 = 2 * p
      j1 = j0 + 1
      cp0 = pltpu.async_copy(y_sh.at[src_v.at[j0]], rows0_v, g0)
      cp1 = pltpu.async_copy(y_sh.at[src_v.at[j1]], rows1_v, g1)
      cp0.wait()
      pltpu.sync_copy(rows0_v, acc_sh.at[dst_v.at[j0]], add=True)
      if with_counts:
        pltpu.sync_copy(ones_v, cnt_sh.at[dst_v.at[j0]], add=True)
      cp1.wait()
      pltpu.sync_copy(rows1_v, acc_sh.at[dst_v.at[j1]], add=True)
      if with_counts:
        pltpu.sync_copy(ones_v, cnt_sh.at[dst_v.at[j1]], add=True)
      return carry

    lax.fori_loop(0, k // 2, pair, 0)
    plsc.subcore_barrier()

    pltpu.sync_copy(acc_sh.at[pl.ds(r0, rpt)], acc_hbm.at[c, pl.ds(r0, rpt)])
    if with_counts:
      pltpu.sync_copy(cnt_sh.at[pl.ds(r0, rpt)], cnt_hbm.at[c, pl.ds(r0, rpt)])

  out_type = [jax.ShapeDtypeStruct((NC, n_pad, h), jnp.float32)]
  scratch = [
      pltpu.VMEM((k, CH), jnp.int32),
      pltpu.VMEM((k, CH), jnp.int32),
      pltpu.VMEM((CH, h), jnp.float32),
      pltpu.VMEM((CH, h), jnp.float32),
  ]
  if with_counts:
    out_type.append(jax.ShapeDtypeStruct((NC, n_pad, CL), jnp.float32))
    scratch.append(pltpu.VMEM((CH, CL), jnp.float32))
  scratch.append(pltpu.VMEM_SHARED((n, h), jnp.float32))
  scratch.append(pltpu.VMEM_SHARED((n_pad, h), jnp.float32))
  if with_counts:
    scratch.append(pltpu.VMEM_SHARED((n_pad, CL), jnp.float32))
  scratch.extend([pltpu.SemaphoreType.DMA] * 4)
  if with_counts:
    scratch.append(pltpu.SemaphoreType.DMA)

  mesh = plsc.VectorSubcoreMesh(
      core_axis_name="c", subcore_axis_name="s",
      num_cores=NC, num_subcores=NS)
  return pl.kernel(
      body, out_type=out_type, mesh=mesh, scratch_types=scratch,
      compiler_params=pltpu.CompilerParams(use_tc_tiling_on_sc=False))


def _tc_in_body(x_ref, wl_ref, wr_ref, y_ref, r_ref):
  xb = x_ref[...]
  y_ref[...] = jnp.dot(xb, wl_ref[...], preferred_element_type=jnp.float32)
  r_ref[...] = jnp.dot(xb, wr_ref[...], preferred_element_type=jnp.float32)


def _relu_mean(acc_ref, cnt_ref, r_ref, b_ref, n):
  cnt = cnt_ref[0, :n, 0:1] + cnt_ref[1, :n, 0:1]
  inv = 1.0 / jnp.maximum(cnt, 1.0)
  agg = acc_ref[0, :n, :] + acc_ref[1, :n, :]
  return jnp.maximum(agg * inv + b_ref[...] + r_ref[...], 0.0)


def _tc_mid_body(n, acc_ref, cnt_ref, r_ref, b_ref, wl_ref, wr_ref,
                 y_ref, rr_ref):
  h = _relu_mean(acc_ref, cnt_ref, r_ref, b_ref, n)
  y_ref[...] = jnp.dot(h, wl_ref[...], preferred_element_type=jnp.float32)
  rr_ref[...] = jnp.dot(h, wr_ref[...], preferred_element_type=jnp.float32)


def _tc_out_body(n, acc_ref, cnt_ref, r_ref, b_ref, wh_ref, bh_ref,
                 h_ref, log_ref):
  h = _relu_mean(acc_ref, cnt_ref, r_ref, b_ref, n)
  h_ref[...] = h
  log_ref[...] = (
      jnp.dot(h, wh_ref[...], preferred_element_type=jnp.float32)
      + bh_ref[...])


def kernel(x, edge_index, W1_l, b1_l, W1_r, W2_l, b2_l, W2_r, Wh, bh):
  n, d = x.shape
  h = W1_l.shape[1]
  out = Wh.shape[1]
  e = edge_index.shape[1]

  n_pad = _cdiv(n + 1, NS * 8) * NS * 8
  k = 2 * _cdiv(_cdiv(e, NW), 2 * CH)
  e_pad = NW * k * CH

  src = edge_index[0].astype(jnp.int32)
  dst = edge_index[1].astype(jnp.int32)
  src3 = jnp.pad(src, (0, e_pad - e)).reshape(NW, k, CH)
  dst3 = jnp.pad(dst, (0, e_pad - e), constant_values=n).reshape(NW, k, CH)

  rpt = n_pad // NS
  zrow = jnp.zeros((rpt, h), jnp.float32)
  zcnt = jnp.zeros((rpt, CL), jnp.float32)
  ones = jnp.ones((CH, CL), jnp.float32)

  sc_scatter1 = _make_sc_scatter(n, n_pad, h, k, with_counts=True)
  sc_scatter2 = _make_sc_scatter(n, n_pad, h, k, with_counts=False)

  tc_in = pl.pallas_call(
      _tc_in_body,
      out_shape=[jax.ShapeDtypeStruct((n, h), jnp.float32),
                 jax.ShapeDtypeStruct((n, h), jnp.float32)])
  tc_mid = pl.pallas_call(
      functools.partial(_tc_mid_body, n),
      out_shape=[jax.ShapeDtypeStruct((n, h), jnp.float32),
                 jax.ShapeDtypeStruct((n, h), jnp.float32)])
  tc_out = pl.pallas_call(
      functools.partial(_tc_out_body, n),
      out_shape=[jax.ShapeDtypeStruct((n, h), jnp.float32),
                 jax.ShapeDtypeStruct((n, out), jnp.float32)])

  y1, r1 = tc_in(x, W1_l, W1_r)
  acc1, cnt = sc_scatter1(y1, src3, dst3, zrow, zcnt, ones)
  y2, r2 = tc_mid(acc1, cnt, r1, b1_l.reshape(1, h), W2_l, W2_r)
  (acc2,) = sc_scatter2(y2, src3, dst3, zrow)
  h2, logits = tc_out(acc2, cnt, r2, b2_l.reshape(1, h), Wh,
                      bh.reshape(1, out))
  return (logits, h2)


# speedup vs baseline: 1.9576x; 1.0198x over previous
import functools

import jax
import jax.numpy as jnp
from jax import lax
from jax.experimental import pallas as pl
from jax.experimental.pallas import tpu as pltpu
from jax.experimental.pallas import tpu_sc as plsc

NC = 2
NS = 16
NW = NC * NS
CH = 128
CL = 16


def _cdiv(a, b):
  return (a + b - 1) // b


def _make_sc_scatter(n, n_pad, h, k, with_counts):
  rpt = n_pad // NS
  assert n % NS == 0

  def body(*refs):
    if with_counts:
      (y_hbm, src_hbm, dst_hbm, zrow_hbm, zcnt_hbm, ones_hbm,
       acc_hbm, cnt_hbm,
       src_v, dst_v, rows0_v, rows1_v, ones_v, y_sh, acc_sh, cnt_sh,
       g0, g1, s0, s1, csem) = refs
    else:
      (y_hbm, src_hbm, dst_hbm, zrow_hbm,
       acc_hbm,
       src_v, dst_v, rows0_v, rows1_v, y_sh, acc_sh,
       g0, g1, s0, s1) = refs

    c = lax.axis_index("c")
    s = lax.axis_index("s")
    wid = c * NS + s
    r0 = s * rpt

    n_rows = y_hbm.shape[0]
    ypt = n_rows // NS
    pltpu.sync_copy(y_hbm.at[pl.ds(s * ypt, ypt)], y_sh.at[pl.ds(s * ypt, ypt)])
    pltpu.sync_copy(zrow_hbm, acc_sh.at[pl.ds(r0, rpt)])
    if with_counts:
      pltpu.sync_copy(zcnt_hbm, cnt_sh.at[pl.ds(r0, rpt)])
      pltpu.sync_copy(ones_hbm, ones_v)

    pltpu.sync_copy(src_hbm.at[wid], src_v)
    pltpu.sync_copy(dst_hbm.at[wid], dst_v)
    plsc.subcore_barrier()

    def pair(p, carry):
      j0 = 2 * p
      j1 = j0 + 1
      cp0 = pltpu.async_copy(y_sh.at[src_v.at[j0]], rows0_v, g0)
      cp1 = pltpu.async_copy(y_sh.at[src_v.at[j1]], rows1_v, g1)
      cp0.wait()
      sc0 = pltpu.async_copy(rows0_v, acc_sh.at[dst_v.at[j0]], s0, add=True)
      if with_counts:
        pltpu.sync_copy(ones_v, cnt_sh.at[dst_v.at[j0]], add=True)
      cp1.wait()
      sc1 = pltpu.async_copy(rows1_v, acc_sh.at[dst_v.at[j1]], s1, add=True)
      if with_counts:
        pltpu.sync_copy(ones_v, cnt_sh.at[dst_v.at[j1]], add=True)
      sc0.wait()
      sc1.wait()
      return carry

    lax.fori_loop(0, k // 2, pair, 0)
    plsc.subcore_barrier()

    pltpu.sync_copy(acc_sh.at[pl.ds(r0, rpt)], acc_hbm.at[c, pl.ds(r0, rpt)])
    if with_counts:
      pltpu.sync_copy(cnt_sh.at[pl.ds(r0, rpt)], cnt_hbm.at[c, pl.ds(r0, rpt)])

  out_type = [jax.ShapeDtypeStruct((NC, n_pad, h), jnp.float32)]
  scratch = [
      pltpu.VMEM((k, CH), jnp.int32),
      pltpu.VMEM((k, CH), jnp.int32),
      pltpu.VMEM((CH, h), jnp.float32),
      pltpu.VMEM((CH, h), jnp.float32),
  ]
  if with_counts:
    out_type.append(jax.ShapeDtypeStruct((NC, n_pad, CL), jnp.float32))
    scratch.append(pltpu.VMEM((CH, CL), jnp.float32))
  scratch.append(pltpu.VMEM_SHARED((n, h), jnp.float32))
  scratch.append(pltpu.VMEM_SHARED((n_pad, h), jnp.float32))
  if with_counts:
    scratch.append(pltpu.VMEM_SHARED((n_pad, CL), jnp.float32))
  scratch.extend([pltpu.SemaphoreType.DMA] * 4)
  if with_counts:
    scratch.append(pltpu.SemaphoreType.DMA)

  mesh = plsc.VectorSubcoreMesh(
      core_axis_name="c", subcore_axis_name="s",
      num_cores=NC, num_subcores=NS)
  return pl.kernel(
      body, out_type=out_type, mesh=mesh, scratch_types=scratch,
      compiler_params=pltpu.CompilerParams(use_tc_tiling_on_sc=False))


def _tc_in_body(x_ref, wl_ref, wr_ref, y_ref, r_ref):
  xb = x_ref[...]
  y_ref[...] = jnp.dot(xb, wl_ref[...], preferred_element_type=jnp.float32)
  r_ref[...] = jnp.dot(xb, wr_ref[...], preferred_element_type=jnp.float32)


def _relu_mean(acc_ref, cnt_ref, r_ref, b_ref, n):
  cnt = cnt_ref[0, :n, 0:1] + cnt_ref[1, :n, 0:1]
  inv = 1.0 / jnp.maximum(cnt, 1.0)
  agg = acc_ref[0, :n, :] + acc_ref[1, :n, :]
  return jnp.maximum(agg * inv + b_ref[...] + r_ref[...], 0.0)


def _tc_mid_body(n, acc_ref, cnt_ref, r_ref, b_ref, wl_ref, wr_ref,
                 y_ref, rr_ref):
  h = _relu_mean(acc_ref, cnt_ref, r_ref, b_ref, n)
  y_ref[...] = jnp.dot(h, wl_ref[...], preferred_element_type=jnp.float32)
  rr_ref[...] = jnp.dot(h, wr_ref[...], preferred_element_type=jnp.float32)


def _tc_out_body(n, acc_ref, cnt_ref, r_ref, b_ref, wh_ref, bh_ref,
                 h_ref, log_ref):
  h = _relu_mean(acc_ref, cnt_ref, r_ref, b_ref, n)
  h_ref[...] = h
  log_ref[...] = (
      jnp.dot(h, wh_ref[...], preferred_element_type=jnp.float32)
      + bh_ref[...])


def kernel(x, edge_index, W1_l, b1_l, W1_r, W2_l, b2_l, W2_r, Wh, bh):
  n, d = x.shape
  h = W1_l.shape[1]
  out = Wh.shape[1]
  e = edge_index.shape[1]

  n_pad = _cdiv(n + 1, NS * 8) * NS * 8
  k = 2 * _cdiv(_cdiv(e, NW), 2 * CH)
  e_pad = NW * k * CH

  src = edge_index[0].astype(jnp.int32)
  dst = edge_index[1].astype(jnp.int32)
  src3 = jnp.pad(src, (0, e_pad - e)).reshape(NW, k, CH)
  dst3 = jnp.pad(dst, (0, e_pad - e), constant_values=n).reshape(NW, k, CH)

  rpt = n_pad // NS
  zrow = jnp.zeros((rpt, h), jnp.float32)
  zcnt = jnp.zeros((rpt, CL), jnp.float32)
  ones = jnp.ones((CH, CL), jnp.float32)

  sc_scatter1 = _make_sc_scatter(n, n_pad, h, k, with_counts=True)
  sc_scatter2 = _make_sc_scatter(n, n_pad, h, k, with_counts=False)

  tc_in = pl.pallas_call(
      _tc_in_body,
      out_shape=[jax.ShapeDtypeStruct((n, h), jnp.float32),
                 jax.ShapeDtypeStruct((n, h), jnp.float32)])
  tc_mid = pl.pallas_call(
      functools.partial(_tc_mid_body, n),
      out_shape=[jax.ShapeDtypeStruct((n, h), jnp.float32),
                 jax.ShapeDtypeStruct((n, h), jnp.float32)])
  tc_out = pl.pallas_call(
      functools.partial(_tc_out_body, n),
      out_shape=[jax.ShapeDtypeStruct((n, h), jnp.float32),
                 jax.ShapeDtypeStruct((n, out), jnp.float32)])

  y1, r1 = tc_in(x, W1_l, W1_r)
  acc1, cnt = sc_scatter1(y1, src3, dst3, zrow, zcnt, ones)
  y2, r2 = tc_mid(acc1, cnt, r1, b1_l.reshape(1, h), W2_l, W2_r)
  (acc2,) = sc_scatter2(y2, src3, dst3, zrow)
  h2, logits = tc_out(acc2, cnt, r2, b2_l.reshape(1, h), Wh,
                      bh.reshape(1, out))
  return (logits, h2)
